# Initial kernel scaffold; baseline (speedup 1.0000x reference)
import functools

import jax
import jax.numpy as jnp
from jax import lax
from jax.experimental import pallas as pl
from jax.experimental.pallas import tpu as pltpu
from jax.experimental.pallas import tpu_sc as plsc

N = 10000
E = 320000
IN_DIM = 128
D1 = 112
D2 = 16
ONES1 = 100
ONES2 = 4

NW = 32
EPW = E // NW
CH = 80
NCH = EPW // CH
ROWS_PER_TILE = N // 16

BLK = 1000
GRID = N // BLK


def _leaky(x):
    return jnp.where(x >= 0.0, x, 0.2 * x)


def _tc1_body(x_ref, w_ref, attT_ref, hp_ref, hpw_ref, av_ref):
    h = jnp.dot(x_ref[...], w_ref[...], preferred_element_type=jnp.float32)
    a2 = jnp.dot(h, attT_ref[...], preferred_element_type=jnp.float32)
    col = lax.broadcasted_iota(jnp.int32, (BLK, D1), 1)
    hp = jnp.where(col == ONES1, 1.0, h)
    wself = jnp.exp(_leaky(a2[:, 0:1] + a2[:, 1:2]))
    hp_ref[...] = hp
    hpw_ref[...] = wself * hp
    av_ref[...] = a2


def _tc1(x, W1p, att1T):
    return pl.pallas_call(
        _tc1_body,
        grid=(GRID,),
        in_specs=[
            pl.BlockSpec((BLK, IN_DIM), lambda i: (i, 0)),
            pl.BlockSpec((IN_DIM, D1), lambda i: (0, 0)),
            pl.BlockSpec((D1, 2), lambda i: (0, 0)),
        ],
        out_specs=[
            pl.BlockSpec((BLK, D1), lambda i: (i, 0)),
            pl.BlockSpec((BLK, D1), lambda i: (i, 0)),
            pl.BlockSpec((BLK, 2), lambda i: (i, 0)),
        ],
        out_shape=[
            jax.ShapeDtypeStruct((N, D1), jnp.float32),
            jax.ShapeDtypeStruct((N, D1), jnp.float32),
            jax.ShapeDtypeStruct((N, 2), jnp.float32),
        ],
    )(x, W1p, att1T)


def _tc2_body(p0_ref, p1_ref, hpw_ref, b1_ref, w2_ref, att2T_ref,
              hp2_ref, hp2w_ref, av2_ref):
    o = p0_ref[...] + p1_ref[...] + hpw_ref[...]
    denom = o[:, ONES1:ONES1 + 1] + 1e-16
    h1 = jnp.maximum(o / denom + b1_ref[...], 0.0)
    h2 = jnp.dot(h1, w2_ref[...], preferred_element_type=jnp.float32)
    a2 = jnp.dot(h2, att2T_ref[...], preferred_element_type=jnp.float32)
    col = lax.broadcasted_iota(jnp.int32, (BLK, D2), 1)
    hp2 = jnp.where(col == ONES2, 1.0, h2)
    wself = jnp.exp(_leaky(a2[:, 0:1] + a2[:, 1:2]))
    hp2_ref[...] = hp2
    hp2w_ref[...] = wself * hp2
    av2_ref[...] = a2


def _tc2(p0, p1, hpw1, b1p, W2p, att2T):
    return pl.pallas_call(
        _tc2_body,
        grid=(GRID,),
        in_specs=[
            pl.BlockSpec((BLK, D1), lambda i: (i, 0)),
            pl.BlockSpec((BLK, D1), lambda i: (i, 0)),
            pl.BlockSpec((BLK, D1), lambda i: (i, 0)),
            pl.BlockSpec((1, D1), lambda i: (0, 0)),
            pl.BlockSpec((D1, D2), lambda i: (0, 0)),
            pl.BlockSpec((D2, 2), lambda i: (0, 0)),
        ],
        out_specs=[
            pl.BlockSpec((BLK, D2), lambda i: (i, 0)),
            pl.BlockSpec((BLK, D2), lambda i: (i, 0)),
            pl.BlockSpec((BLK, 2), lambda i: (i, 0)),
        ],
        out_shape=[
            jax.ShapeDtypeStruct((N, D2), jnp.float32),
            jax.ShapeDtypeStruct((N, D2), jnp.float32),
            jax.ShapeDtypeStruct((N, 2), jnp.float32),
        ],
    )(p0, p1, hpw1, b1p, W2p, att2T)


def _tc3_body(q0_ref, q1_ref, hp2w_ref, b2_ref, out_ref):
    o = q0_ref[...] + q1_ref[...] + hp2w_ref[...]
    denom = o[:, ONES2:ONES2 + 1] + 1e-16
    logits = o / denom + b2_ref[...]
    col = lax.broadcasted_iota(jnp.int32, (BLK, D2), 1)
    valid = col < ONES2
    ml = jnp.where(valid, logits, -1e30)
    m = jnp.max(ml, axis=1, keepdims=True)
    s = jnp.sum(jnp.where(valid, jnp.exp(ml - m), 0.0), axis=1, keepdims=True)
    out_ref[...] = (logits - m - jnp.log(s))[:, 0:ONES2]


def _tc3(q0, q1, hp2w, b2p):
    return pl.pallas_call(
        _tc3_body,
        grid=(GRID,),
        in_specs=[
            pl.BlockSpec((BLK, D2), lambda i: (i, 0)),
            pl.BlockSpec((BLK, D2), lambda i: (i, 0)),
            pl.BlockSpec((BLK, D2), lambda i: (i, 0)),
            pl.BlockSpec((1, D2), lambda i: (0, 0)),
        ],
        out_specs=[pl.BlockSpec((BLK, ONES2), lambda i: (i, 0))],
        out_shape=[jax.ShapeDtypeStruct((N, ONES2), jnp.float32)],
    )(q0, q1, hp2w, b2p)


def _sc_edge_body(D, hp_hbm, src_hbm, dst_hbm, asrc_hbm, adst_hbm, zeros_hbm,
                  out_hbm, accum, rows_v, src_v, dst_v, w_v, asrc_v, adst_v,
                  sem):
    cid = lax.axis_index("c")
    sid = lax.axis_index("s")
    wid = sid * 2 + cid

    pltpu.sync_copy(asrc_hbm, asrc_v)
    pltpu.sync_copy(adst_hbm, adst_v)
    pltpu.sync_copy(zeros_hbm, accum.at[pl.ds(sid * ROWS_PER_TILE,
                                              ROWS_PER_TILE)])
    plsc.subcore_barrier()

    def chunk(c, carry):
        base = wid * EPW + c * CH
        pltpu.sync_copy(src_hbm.at[pl.ds(base, CH)], src_v)
        pltpu.sync_copy(dst_hbm.at[pl.ds(base, CH)], dst_v)
        cp = pltpu.async_copy(hp_hbm.at[src_v], rows_v, sem)
        for g in range(CH // 16):
            sv = src_v[pl.ds(g * 16, 16)]
            dv = dst_v[pl.ds(g * 16, 16)]
            e = (plsc.load_gather(asrc_v, [sv])
                 + plsc.load_gather(adst_v, [dv]))
            e = jnp.where(e >= 0.0, e, 0.2 * e)
            w_v[pl.ds(g * 16, 16)] = jnp.exp(e)
        cp.wait()

        def scale(i, c2):
            wi = w_v[i]
            for j in range(D // 16):
                rows_v[i, pl.ds(j * 16, 16)] = (
                    rows_v[i, pl.ds(j * 16, 16)] * wi)
            return c2

        lax.fori_loop(0, CH, scale, 0)
        pltpu.sync_copy(rows_v, accum.at[dst_v], add=True)
        return carry

    lax.fori_loop(0, NCH, chunk, 0)
    plsc.subcore_barrier()
    pltpu.sync_copy(
        accum.at[pl.ds(sid * ROWS_PER_TILE, ROWS_PER_TILE)],
        out_hbm.at[cid].at[pl.ds(sid * ROWS_PER_TILE, ROWS_PER_TILE)])


def _sc_edge(D, hp, src, dst, asrc, adst):
    zeros = jnp.zeros((ROWS_PER_TILE, D), jnp.float32)
    mesh = plsc.VectorSubcoreMesh(core_axis_name="c", subcore_axis_name="s")
    return pl.kernel(
        functools.partial(_sc_edge_body, D),
        out_type=jax.ShapeDtypeStruct((2, N, D), jnp.float32),
        mesh=mesh,
        scratch_types=[
            pltpu.VMEM_SHARED((N, D), jnp.float32),
            pltpu.VMEM((CH, D), jnp.float32),
            pltpu.VMEM((CH,), jnp.int32),
            pltpu.VMEM((CH,), jnp.int32),
            pltpu.VMEM((CH,), jnp.float32),
            pltpu.VMEM((N,), jnp.float32),
            pltpu.VMEM((N,), jnp.float32),
            pltpu.SemaphoreType.DMA,
        ],
    )(hp, src, dst, asrc, adst, zeros)


def kernel(x, edge_index, W1, att_src1, att_dst1, b1, W2, att_src2, att_dst2,
           b2):
    src = edge_index[0].astype(jnp.int32)
    dst = edge_index[1].astype(jnp.int32)

    W1p = jnp.zeros((IN_DIM, D1), jnp.float32).at[:, :100].set(W1)
    att1T = jnp.zeros((D1, 2), jnp.float32)
    att1T = att1T.at[:100, 0].set(att_src1).at[:100, 1].set(att_dst1)
    b1p = jnp.zeros((1, D1), jnp.float32).at[0, :100].set(b1)
    W2p = jnp.zeros((D1, D2), jnp.float32).at[:100, :4].set(W2)
    att2T = jnp.zeros((D2, 2), jnp.float32)
    att2T = att2T.at[:4, 0].set(att_src2).at[:4, 1].set(att_dst2)
    b2p = jnp.zeros((1, D2), jnp.float32).at[0, :4].set(b2)

    hp1, hpw1, av1 = _tc1(x, W1p, att1T)
    part1 = _sc_edge(D1, hp1, src, dst,
                     jnp.ascontiguousarray(av1[:, 0]),
                     jnp.ascontiguousarray(av1[:, 1]))
    hp2, hp2w, av2 = _tc2(part1[0], part1[1], hpw1, b1p, W2p, att2T)
    part2 = _sc_edge(D2, hp2, src, dst,
                     jnp.ascontiguousarray(av2[:, 0]),
                     jnp.ascontiguousarray(av2[:, 1]))
    (out,) = _tc3(part2[0], part2[1], hp2w, b2p)
    return out


# speedup vs baseline: 32.0757x; 32.0757x over previous
import functools

import jax
import jax.numpy as jnp
from jax import lax
from jax.experimental import pallas as pl
from jax.experimental.pallas import tpu as pltpu
from jax.experimental.pallas import tpu_sc as plsc

N = 10000
N_PAD = 10112
E = 320000
IN_DIM = 128
D1 = 112
D2 = 16
ONES1 = 100
ONES2 = 4

NW = 32
EPW = E // NW
CH = 80
NCH = EPW // CH
ROWS_PER_TILE = N_PAD // 16

BLK = 1264
GRID = N_PAD // BLK


def _leaky(x):
    return jnp.where(x >= 0.0, x, 0.2 * x)


def _tc1_body(x_ref, w_ref, attT_ref, hp_ref, hpw_ref, av_ref):
    h = jnp.dot(x_ref[...], w_ref[...], preferred_element_type=jnp.float32)
    a2 = jnp.dot(h, attT_ref[...], preferred_element_type=jnp.float32)
    col = lax.broadcasted_iota(jnp.int32, (BLK, D1), 1)
    hp = jnp.where(col == ONES1, 1.0, h)
    wself = jnp.exp(_leaky(a2[:, 0:1] + a2[:, 1:2]))
    hp_ref[...] = hp
    hpw_ref[...] = wself * hp
    av_ref[...] = a2


def _tc1(x, W1p, att1T):
    return pl.pallas_call(
        _tc1_body,
        grid=(GRID,),
        in_specs=[
            pl.BlockSpec((BLK, IN_DIM), lambda i: (i, 0)),
            pl.BlockSpec((IN_DIM, D1), lambda i: (0, 0)),
            pl.BlockSpec((D1, 2), lambda i: (0, 0)),
        ],
        out_specs=[
            pl.BlockSpec((BLK, D1), lambda i: (i, 0)),
            pl.BlockSpec((BLK, D1), lambda i: (i, 0)),
            pl.BlockSpec((BLK, 2), lambda i: (i, 0)),
        ],
        out_shape=[
            jax.ShapeDtypeStruct((N_PAD, D1), jnp.float32),
            jax.ShapeDtypeStruct((N_PAD, D1), jnp.float32),
            jax.ShapeDtypeStruct((N_PAD, 2), jnp.float32),
        ],
    )(x, W1p, att1T)


def _tc2_body(p0_ref, p1_ref, hpw_ref, b1_ref, w2_ref, att2T_ref,
              hp2_ref, hp2w_ref, av2_ref):
    o = p0_ref[...] + p1_ref[...] + hpw_ref[...]
    denom = o[:, ONES1:ONES1 + 1] + 1e-16
    h1 = jnp.maximum(o / denom + b1_ref[...], 0.0)
    h2 = jnp.dot(h1, w2_ref[...], preferred_element_type=jnp.float32)
    a2 = jnp.dot(h2, att2T_ref[...], preferred_element_type=jnp.float32)
    col = lax.broadcasted_iota(jnp.int32, (BLK, D2), 1)
    hp2 = jnp.where(col == ONES2, 1.0, h2)
    wself = jnp.exp(_leaky(a2[:, 0:1] + a2[:, 1:2]))
    hp2_ref[...] = hp2
    hp2w_ref[...] = wself * hp2
    av2_ref[...] = a2


def _tc2(p0, p1, hpw1, b1p, W2p, att2T):
    return pl.pallas_call(
        _tc2_body,
        grid=(GRID,),
        in_specs=[
            pl.BlockSpec((BLK, D1), lambda i: (i, 0)),
            pl.BlockSpec((BLK, D1), lambda i: (i, 0)),
            pl.BlockSpec((BLK, D1), lambda i: (i, 0)),
            pl.BlockSpec((1, D1), lambda i: (0, 0)),
            pl.BlockSpec((D1, D2), lambda i: (0, 0)),
            pl.BlockSpec((D2, 2), lambda i: (0, 0)),
        ],
        out_specs=[
            pl.BlockSpec((BLK, D2), lambda i: (i, 0)),
            pl.BlockSpec((BLK, D2), lambda i: (i, 0)),
            pl.BlockSpec((BLK, 2), lambda i: (i, 0)),
        ],
        out_shape=[
            jax.ShapeDtypeStruct((N_PAD, D2), jnp.float32),
            jax.ShapeDtypeStruct((N_PAD, D2), jnp.float32),
            jax.ShapeDtypeStruct((N_PAD, 2), jnp.float32),
        ],
    )(p0, p1, hpw1, b1p, W2p, att2T)


def _tc3_body(q0_ref, q1_ref, hp2w_ref, b2_ref, out_ref):
    o = q0_ref[...] + q1_ref[...] + hp2w_ref[...]
    denom = o[:, ONES2:ONES2 + 1] + 1e-16
    logits = o / denom + b2_ref[...]
    col = lax.broadcasted_iota(jnp.int32, (BLK, D2), 1)
    valid = col < ONES2
    ml = jnp.where(valid, logits, -1e30)
    m = jnp.max(ml, axis=1, keepdims=True)
    s = jnp.sum(jnp.where(valid, jnp.exp(ml - m), 0.0), axis=1, keepdims=True)
    out_ref[...] = (logits - m - jnp.log(s))[:, 0:ONES2]


def _tc3(q0, q1, hp2w, b2p):
    return pl.pallas_call(
        _tc3_body,
        grid=(GRID,),
        in_specs=[
            pl.BlockSpec((BLK, D2), lambda i: (i, 0)),
            pl.BlockSpec((BLK, D2), lambda i: (i, 0)),
            pl.BlockSpec((BLK, D2), lambda i: (i, 0)),
            pl.BlockSpec((1, D2), lambda i: (0, 0)),
        ],
        out_specs=[pl.BlockSpec((BLK, ONES2), lambda i: (i, 0))],
        out_shape=[jax.ShapeDtypeStruct((N_PAD, ONES2), jnp.float32)],
    )(q0, q1, hp2w, b2p)


def _sc_edge_body(D, hp_hbm, src_hbm, dst_hbm, asrc_hbm, adst_hbm, zeros_hbm,
                  out_hbm, accum, rows_v, src_v, dst_v, asrc_v, adst_v,
                  sem):
    cid = lax.axis_index("c")
    sid = lax.axis_index("s")
    wid = sid * 2 + cid

    pltpu.sync_copy(asrc_hbm, asrc_v)
    pltpu.sync_copy(adst_hbm, adst_v)
    pltpu.sync_copy(zeros_hbm, accum.at[pl.ds(sid * ROWS_PER_TILE,
                                              ROWS_PER_TILE)])
    plsc.subcore_barrier()

    def chunk(c, carry):
        base = wid * EPW + c * CH
        pltpu.sync_copy(src_hbm.at[pl.ds(base, CH)], src_v)
        pltpu.sync_copy(dst_hbm.at[pl.ds(base, CH)], dst_v)
        cp = pltpu.async_copy(hp_hbm.at[src_v], rows_v, sem)
        ws = []
        for g in range(CH // 16):
            sv = src_v[pl.ds(g * 16, 16)]
            dv = dst_v[pl.ds(g * 16, 16)]
            e = (plsc.load_gather(asrc_v, [sv])
                 + plsc.load_gather(adst_v, [dv]))
            e = jnp.where(e >= 0.0, e, 0.2 * e)
            ws.append(jnp.exp(e))
        cp.wait()
        for g in range(CH // 16):
            for l in range(16):
                wl = ws[g][l]
                i = g * 16 + l
                for j in range(D // 16):
                    rows_v[i, pl.ds(j * 16, 16)] = (
                        rows_v[i, pl.ds(j * 16, 16)] * wl)
        pltpu.sync_copy(rows_v, accum.at[dst_v], add=True)
        return carry

    lax.fori_loop(0, NCH, chunk, 0)
    plsc.subcore_barrier()
    pltpu.sync_copy(
        accum.at[pl.ds(sid * ROWS_PER_TILE, ROWS_PER_TILE)],
        out_hbm.at[cid].at[pl.ds(sid * ROWS_PER_TILE, ROWS_PER_TILE)])


def _sc_edge(D, hp, src, dst, asrc, adst):
    zeros = jnp.zeros((ROWS_PER_TILE, D), jnp.float32)
    mesh = plsc.VectorSubcoreMesh(core_axis_name="c", subcore_axis_name="s")
    return pl.kernel(
        functools.partial(_sc_edge_body, D),
        out_type=jax.ShapeDtypeStruct((2, N_PAD, D), jnp.float32),
        mesh=mesh,
        scratch_types=[
            pltpu.VMEM_SHARED((N_PAD, D), jnp.float32),
            pltpu.VMEM((CH, D), jnp.float32),
            pltpu.VMEM((CH,), jnp.int32),
            pltpu.VMEM((CH,), jnp.int32),
            pltpu.VMEM((N_PAD,), jnp.float32),
            pltpu.VMEM((N_PAD,), jnp.float32),
            pltpu.SemaphoreType.DMA,
        ],
        compiler_params=pltpu.CompilerParams(needs_layout_passes=False,
                                             use_tc_tiling_on_sc=False),
    )(hp, src, dst, asrc, adst, zeros)


def kernel(x, edge_index, W1, att_src1, att_dst1, b1, W2, att_src2, att_dst2,
           b2):
    src = edge_index[0].astype(jnp.int32)
    dst = edge_index[1].astype(jnp.int32)

    W1p = jnp.zeros((IN_DIM, D1), jnp.float32).at[:, :100].set(W1)
    att1T = jnp.zeros((D1, 2), jnp.float32)
    att1T = att1T.at[:100, 0].set(att_src1).at[:100, 1].set(att_dst1)
    b1p = jnp.zeros((1, D1), jnp.float32).at[0, :100].set(b1)
    W2p = jnp.zeros((D1, D2), jnp.float32).at[:100, :4].set(W2)
    att2T = jnp.zeros((D2, 2), jnp.float32)
    att2T = att2T.at[:4, 0].set(att_src2).at[:4, 1].set(att_dst2)
    b2p = jnp.zeros((1, D2), jnp.float32).at[0, :4].set(b2)

    xp = jnp.zeros((N_PAD, IN_DIM), jnp.float32).at[:N].set(x)
    hp1, hpw1, av1 = _tc1(xp, W1p, att1T)
    part1 = _sc_edge(D1, hp1, src, dst,
                     av1[:, 0] + 0.0, av1[:, 1] + 0.0)
    hp2, hp2w, av2 = _tc2(part1[0], part1[1], hpw1, b1p, W2p, att2T)
    part2 = _sc_edge(D2, hp2, src, dst,
                     av2[:, 0] + 0.0, av2[:, 1] + 0.0)
    (out,) = _tc3(part2[0], part2[1], hp2w, b2p)
    return out[:N]


# speedup vs baseline: 48.5825x; 1.5146x over previous
import functools

import jax
import jax.numpy as jnp
from jax import lax
from jax.experimental import pallas as pl
from jax.experimental.pallas import tpu as pltpu
from jax.experimental.pallas import tpu_sc as plsc

N = 10000
N_PAD = 10112
E = 320000
IN_DIM = 128
D1 = 112
D2 = 16
ONES1 = 100
ONES2 = 4

NW = 32
CH = 64
NG = CH // 16
NCH = 158
E_PAD = NW * NCH * CH
PK = 16384
ROWS_PER_TILE = N_PAD // 16

BLK = 1264
GRID = N_PAD // BLK


def _leaky(x):
    return jnp.where(x >= 0.0, x, 0.2 * x)


def _tc1_body(x_ref, w_ref, attT_ref, hp_ref, hpw_ref, av_ref):
    h = jnp.dot(x_ref[...], w_ref[...], preferred_element_type=jnp.float32)
    a2 = jnp.dot(h, attT_ref[...], preferred_element_type=jnp.float32)
    col = lax.broadcasted_iota(jnp.int32, (BLK, D1), 1)
    hp = jnp.where(col == ONES1, 1.0, h)
    wself = jnp.exp(_leaky(a2[:, 0:1] + a2[:, 1:2]))
    hp_ref[...] = hp
    hpw_ref[...] = wself * hp
    av_ref[...] = a2


def _tc1(x, W1p, att1T):
    return pl.pallas_call(
        _tc1_body,
        grid=(GRID,),
        in_specs=[
            pl.BlockSpec((BLK, IN_DIM), lambda i: (i, 0)),
            pl.BlockSpec((IN_DIM, D1), lambda i: (0, 0)),
            pl.BlockSpec((D1, 2), lambda i: (0, 0)),
        ],
        out_specs=[
            pl.BlockSpec((BLK, D1), lambda i: (i, 0)),
            pl.BlockSpec((BLK, D1), lambda i: (i, 0)),
            pl.BlockSpec((BLK, 2), lambda i: (i, 0)),
        ],
        out_shape=[
            jax.ShapeDtypeStruct((N_PAD, D1), jnp.float32),
            jax.ShapeDtypeStruct((N_PAD, D1), jnp.float32),
            jax.ShapeDtypeStruct((N_PAD, 2), jnp.float32),
        ],
    )(x, W1p, att1T)


def _tc2_body(p0_ref, p1_ref, hpw_ref, b1_ref, w2_ref, att2T_ref,
              hp2_ref, hp2w_ref, av2_ref):
    o = p0_ref[...] + p1_ref[...] + hpw_ref[...]
    denom = o[:, ONES1:ONES1 + 1] + 1e-16
    h1 = jnp.maximum(o / denom + b1_ref[...], 0.0)
    h2 = jnp.dot(h1, w2_ref[...], preferred_element_type=jnp.float32)
    a2 = jnp.dot(h2, att2T_ref[...], preferred_element_type=jnp.float32)
    col = lax.broadcasted_iota(jnp.int32, (BLK, D2), 1)
    hp2 = jnp.where(col == ONES2, 1.0, h2)
    wself = jnp.exp(_leaky(a2[:, 0:1] + a2[:, 1:2]))
    hp2_ref[...] = hp2
    hp2w_ref[...] = wself * hp2
    av2_ref[...] = a2


def _tc2(p0, p1, hpw1, b1p, W2p, att2T):
    return pl.pallas_call(
        _tc2_body,
        grid=(GRID,),
        in_specs=[
            pl.BlockSpec((BLK, D1), lambda i: (i, 0)),
            pl.BlockSpec((BLK, D1), lambda i: (i, 0)),
            pl.BlockSpec((BLK, D1), lambda i: (i, 0)),
            pl.BlockSpec((1, D1), lambda i: (0, 0)),
            pl.BlockSpec((D1, D2), lambda i: (0, 0)),
            pl.BlockSpec((D2, 2), lambda i: (0, 0)),
        ],
        out_specs=[
            pl.BlockSpec((BLK, D2), lambda i: (i, 0)),
            pl.BlockSpec((BLK, D2), lambda i: (i, 0)),
            pl.BlockSpec((BLK, 2), lambda i: (i, 0)),
        ],
        out_shape=[
            jax.ShapeDtypeStruct((N_PAD, D2), jnp.float32),
            jax.ShapeDtypeStruct((N_PAD, D2), jnp.float32),
            jax.ShapeDtypeStruct((N_PAD, 2), jnp.float32),
        ],
    )(p0, p1, hpw1, b1p, W2p, att2T)


def _tc3_body(q0_ref, q1_ref, hp2w_ref, b2_ref, out_ref):
    o = q0_ref[...] + q1_ref[...] + hp2w_ref[...]
    denom = o[:, ONES2:ONES2 + 1] + 1e-16
    logits = o / denom + b2_ref[...]
    col = lax.broadcasted_iota(jnp.int32, (BLK, D2), 1)
    valid = col < ONES2
    ml = jnp.where(valid, logits, -1e30)
    m = jnp.max(ml, axis=1, keepdims=True)
    s = jnp.sum(jnp.where(valid, jnp.exp(ml - m), 0.0), axis=1, keepdims=True)
    out_ref[...] = (logits - m - jnp.log(s))[:, 0:ONES2]


def _tc3(q0, q1, hp2w, b2p):
    return pl.pallas_call(
        _tc3_body,
        grid=(GRID,),
        in_specs=[
            pl.BlockSpec((BLK, D2), lambda i: (i, 0)),
            pl.BlockSpec((BLK, D2), lambda i: (i, 0)),
            pl.BlockSpec((BLK, D2), lambda i: (i, 0)),
            pl.BlockSpec((1, D2), lambda i: (0, 0)),
        ],
        out_specs=[pl.BlockSpec((BLK, ONES2), lambda i: (i, 0))],
        out_shape=[jax.ShapeDtypeStruct((N_PAD, ONES2), jnp.float32)],
    )(q0, q1, hp2w, b2p)


def _sc_edge_body(D, hp_hbm, pk_hbm, asrc_hbm, adst_hbm, zeros_hbm,
                  out_hbm, accum, rows0, rows1, sbuf0, sbuf1, pkx,
                  su0, su1, du0, du1, asrc_v, adst_v, g0, g1, s0, s1):
    cid = lax.axis_index("c")
    sid = lax.axis_index("s")
    wid = sid * 2 + cid

    pltpu.sync_copy(pk_hbm.at[wid], pkx)
    pltpu.sync_copy(asrc_hbm, asrc_v)
    pltpu.sync_copy(adst_hbm, adst_v)
    pltpu.sync_copy(zeros_hbm, accum.at[pl.ds(sid * ROWS_PER_TILE,
                                              ROWS_PER_TILE)])
    plsc.subcore_barrier()

    def unpack(c):
        svs, dvs = [], []
        for g in range(NG):
            pk = pkx[c, pl.ds(g * 16, 16)]
            svs.append(lax.shift_right_logical(pk, 14))
            dvs.append(lax.bitwise_and(pk, PK - 1))
        return svs, dvs

    def write_src(c, su):
        svs, _ = unpack(c)
        for g in range(NG):
            su[pl.ds(g * 16, 16)] = svs[g]

    def process(c, rows, sbuf, su, du, gsem, ssem, wait_scatter,
                prefetch_guarded):
        pltpu.make_async_copy(hp_hbm.at[su], rows, gsem).wait()
        svs, dvs = unpack(c)
        ws = []
        for g in range(NG):
            e = (plsc.load_gather(asrc_v, [svs[g]])
                 + plsc.load_gather(adst_v, [dvs[g]]))
            e = jnp.where(e >= 0.0, e, 0.2 * e)
            ws.append(jnp.exp(e))
        if wait_scatter:
            pltpu.make_async_copy(sbuf, accum.at[du], ssem).wait()
        for g in range(NG):
            du[pl.ds(g * 16, 16)] = dvs[g]
        for g in range(NG):
            for l in range(16):
                wl = ws[g][l]
                i = g * 16 + l
                for j in range(D // 16):
                    sbuf[i, pl.ds(j * 16, 16)] = (
                        rows[i, pl.ds(j * 16, 16)] * wl)
        def prefetch():
            write_src(c + 2, su)
            pltpu.async_copy(hp_hbm.at[su], rows, gsem)
        if prefetch_guarded:
            pl.when(c + 2 < NCH)(prefetch)
        else:
            prefetch()
        pltpu.async_copy(sbuf, accum.at[du], ssem, add=True)

    write_src(0, su0)
    write_src(1, su1)
    pltpu.async_copy(hp_hbm.at[su0], rows0, g0)
    pltpu.async_copy(hp_hbm.at[su1], rows1, g1)
    process(0, rows0, sbuf0, su0, du0, g0, s0, False, False)
    process(1, rows1, sbuf1, su1, du1, g1, s1, False, False)

    def pair(k, carry):
        process(2 * k, rows0, sbuf0, su0, du0, g0, s0, True, True)
        process(2 * k + 1, rows1, sbuf1, su1, du1, g1, s1, True, True)
        return carry

    lax.fori_loop(1, NCH // 2, pair, 0)
    pltpu.make_async_copy(sbuf0, accum.at[du0], s0).wait()
    pltpu.make_async_copy(sbuf1, accum.at[du1], s1).wait()
    plsc.subcore_barrier()
    pltpu.sync_copy(
        accum.at[pl.ds(sid * ROWS_PER_TILE, ROWS_PER_TILE)],
        out_hbm.at[cid].at[pl.ds(sid * ROWS_PER_TILE, ROWS_PER_TILE)])


def _sc_edge(D, hp, pk, asrc, adst):
    zeros = jnp.zeros((ROWS_PER_TILE, D), jnp.float32)
    mesh = plsc.VectorSubcoreMesh(core_axis_name="c", subcore_axis_name="s")
    return pl.kernel(
        functools.partial(_sc_edge_body, D),
        out_type=jax.ShapeDtypeStruct((2, N_PAD, D), jnp.float32),
        mesh=mesh,
        scratch_types=[
            pltpu.VMEM_SHARED((N_PAD, D), jnp.float32),
            pltpu.VMEM((CH, D), jnp.float32),
            pltpu.VMEM((CH, D), jnp.float32),
            pltpu.VMEM((CH, D), jnp.float32),
            pltpu.VMEM((CH, D), jnp.float32),
            pltpu.VMEM((NCH, CH), jnp.int32),
            pltpu.VMEM((CH,), jnp.int32),
            pltpu.VMEM((CH,), jnp.int32),
            pltpu.VMEM((CH,), jnp.int32),
            pltpu.VMEM((CH,), jnp.int32),
            pltpu.VMEM((N_PAD,), jnp.float32),
            pltpu.VMEM((N_PAD,), jnp.float32),
            pltpu.SemaphoreType.DMA,
            pltpu.SemaphoreType.DMA,
            pltpu.SemaphoreType.DMA,
            pltpu.SemaphoreType.DMA,
        ],
        compiler_params=pltpu.CompilerParams(needs_layout_passes=False,
                                             use_tc_tiling_on_sc=False),
    )(hp, pk, asrc, adst, zeros)


def kernel(x, edge_index, W1, att_src1, att_dst1, b1, W2, att_src2, att_dst2,
           b2):
    pad = jnp.full((E_PAD - E,), N * PK + N, jnp.int32)
    pk = jnp.concatenate(
        [(edge_index[0] * PK + edge_index[1]).astype(jnp.int32), pad]
    ).reshape(NW, NCH, CH)

    W1p = jnp.zeros((IN_DIM, D1), jnp.float32).at[:, :100].set(W1)
    att1T = jnp.zeros((D1, 2), jnp.float32)
    att1T = att1T.at[:100, 0].set(att_src1).at[:100, 1].set(att_dst1)
    b1p = jnp.zeros((1, D1), jnp.float32).at[0, :100].set(b1)
    W2p = jnp.zeros((D1, D2), jnp.float32).at[:100, :4].set(W2)
    att2T = jnp.zeros((D2, 2), jnp.float32)
    att2T = att2T.at[:4, 0].set(att_src2).at[:4, 1].set(att_dst2)
    b2p = jnp.zeros((1, D2), jnp.float32).at[0, :4].set(b2)

    xp = jnp.zeros((N_PAD, IN_DIM), jnp.float32).at[:N].set(x)
    hp1, hpw1, av1 = _tc1(xp, W1p, att1T)
    part1 = _sc_edge(D1, hp1, pk, av1[:, 0] + 0.0, av1[:, 1] + 0.0)
    hp2, hp2w, av2 = _tc2(part1[0], part1[1], hpw1, b1p, W2p, att2T)
    part2 = _sc_edge(D2, hp2, pk, av2[:, 0] + 0.0, av2[:, 1] + 0.0)
    (out,) = _tc3(part2[0], part2[1], hp2w, b2p)
    return out[:N]


# speedup vs baseline: 56.4567x; 1.1621x over previous
import functools

import jax
import jax.numpy as jnp
import numpy as np
from jax import lax
from jax.experimental import pallas as pl
from jax.experimental.pallas import tpu as pltpu
from jax.experimental.pallas import tpu_sc as plsc

N = 10000
N_PAD = 10112
E = 320000
IN_DIM = 128
D1 = 112
D2 = 16
ONES1 = 100
ONES2 = 4

NW = 32
CH = 96
NG = CH // 16
NCH = 106
E_PAD = NW * NCH * CH
DB = 128
PK = 16384
ROWS_PER_TILE = N_PAD // 16

BLK = 1264
GRID = N_PAD // BLK


def _leaky(x):
    return jnp.where(x >= 0.0, x, 0.2 * x)


def _tc1_body(x_ref, w_ref, attT_ref, pmat_ref, hp_ref, hpw_ref, av_ref):
    h = jnp.dot(x_ref[...], w_ref[...], preferred_element_type=jnp.float32)
    a2 = jnp.dot(h, attT_ref[...], preferred_element_type=jnp.float32)
    col = lax.broadcasted_iota(jnp.int32, (BLK, D1), 1)
    hp = jnp.where(col == ONES1, 1.0, h)
    wself = jnp.exp(_leaky(a2[:, 0:1] + a2[:, 1:2]))
    hps = jnp.dot(hp, pmat_ref[...], preferred_element_type=jnp.float32)
    hp_ref[...] = hps.astype(jnp.bfloat16)
    hpw_ref[...] = wself * hp
    av_ref[...] = a2


def _tc1(x, W1p, att1T, Pmat):
    return pl.pallas_call(
        _tc1_body,
        grid=(GRID,),
        in_specs=[
            pl.BlockSpec((BLK, IN_DIM), lambda i: (i, 0)),
            pl.BlockSpec((IN_DIM, D1), lambda i: (0, 0)),
            pl.BlockSpec((D1, 2), lambda i: (0, 0)),
            pl.BlockSpec((D1, DB), lambda i: (0, 0)),
        ],
        out_specs=[
            pl.BlockSpec((BLK, DB), lambda i: (i, 0)),
            pl.BlockSpec((BLK, D1), lambda i: (i, 0)),
            pl.BlockSpec((BLK, 2), lambda i: (i, 0)),
        ],
        out_shape=[
            jax.ShapeDtypeStruct((N_PAD, DB), jnp.bfloat16),
            jax.ShapeDtypeStruct((N_PAD, D1), jnp.float32),
            jax.ShapeDtypeStruct((N_PAD, 2), jnp.float32),
        ],
    )(x, W1p, att1T, Pmat)


def _tc2_body(p0_ref, p1_ref, hpw_ref, b1_ref, w2_ref, att2T_ref,
              hp2_ref, hp2w_ref, av2_ref):
    o = p0_ref[...] + p1_ref[...] + hpw_ref[...]
    denom = o[:, ONES1:ONES1 + 1] + 1e-16
    h1 = jnp.maximum(o / denom + b1_ref[...], 0.0)
    h2 = jnp.dot(h1, w2_ref[...], preferred_element_type=jnp.float32)
    a2 = jnp.dot(h2, att2T_ref[...], preferred_element_type=jnp.float32)
    col = lax.broadcasted_iota(jnp.int32, (BLK, D2), 1)
    hp2 = jnp.where(col == ONES2, 1.0, h2)
    wself = jnp.exp(_leaky(a2[:, 0:1] + a2[:, 1:2]))
    hp2_ref[...] = hp2
    hp2w_ref[...] = wself * hp2
    av2_ref[...] = a2


def _tc2(p0, p1, hpw1, b1p, W2p, att2T):
    return pl.pallas_call(
        _tc2_body,
        grid=(GRID,),
        in_specs=[
            pl.BlockSpec((BLK, D1), lambda i: (i, 0)),
            pl.BlockSpec((BLK, D1), lambda i: (i, 0)),
            pl.BlockSpec((BLK, D1), lambda i: (i, 0)),
            pl.BlockSpec((1, D1), lambda i: (0, 0)),
            pl.BlockSpec((D1, D2), lambda i: (0, 0)),
            pl.BlockSpec((D2, 2), lambda i: (0, 0)),
        ],
        out_specs=[
            pl.BlockSpec((BLK, D2), lambda i: (i, 0)),
            pl.BlockSpec((BLK, D2), lambda i: (i, 0)),
            pl.BlockSpec((BLK, 2), lambda i: (i, 0)),
        ],
        out_shape=[
            jax.ShapeDtypeStruct((N_PAD, D2), jnp.float32),
            jax.ShapeDtypeStruct((N_PAD, D2), jnp.float32),
            jax.ShapeDtypeStruct((N_PAD, 2), jnp.float32),
        ],
    )(p0, p1, hpw1, b1p, W2p, att2T)


def _tc3_body(q0_ref, q1_ref, hp2w_ref, b2_ref, out_ref):
    o = q0_ref[...] + q1_ref[...] + hp2w_ref[...]
    denom = o[:, ONES2:ONES2 + 1] + 1e-16
    logits = o / denom + b2_ref[...]
    col = lax.broadcasted_iota(jnp.int32, (BLK, D2), 1)
    valid = col < ONES2
    ml = jnp.where(valid, logits, -1e30)
    m = jnp.max(ml, axis=1, keepdims=True)
    s = jnp.sum(jnp.where(valid, jnp.exp(ml - m), 0.0), axis=1, keepdims=True)
    out_ref[...] = (logits - m - jnp.log(s))[:, 0:ONES2]


def _tc3(q0, q1, hp2w, b2p):
    return pl.pallas_call(
        _tc3_body,
        grid=(GRID,),
        in_specs=[
            pl.BlockSpec((BLK, D2), lambda i: (i, 0)),
            pl.BlockSpec((BLK, D2), lambda i: (i, 0)),
            pl.BlockSpec((BLK, D2), lambda i: (i, 0)),
            pl.BlockSpec((1, D2), lambda i: (0, 0)),
        ],
        out_specs=[pl.BlockSpec((BLK, ONES2), lambda i: (i, 0))],
        out_shape=[jax.ShapeDtypeStruct((N_PAD, ONES2), jnp.float32)],
    )(q0, q1, hp2w, b2p)


def _sc_edge_body(D, bf16_rows, hp_hbm, pk_hbm, apair_hbm, zeros_hbm,
                  out_hbm, accum, rows0, rows1, sbuf0, sbuf1, pkx,
                  su0, su1, du0, du1, apair_v, g0, g1, s0, s1):
    cid = lax.axis_index("c")
    sid = lax.axis_index("s")
    wid = sid * 2 + cid

    pltpu.sync_copy(pk_hbm.at[wid], pkx)
    pltpu.sync_copy(apair_hbm, apair_v)
    pltpu.sync_copy(zeros_hbm, accum.at[pl.ds(sid * ROWS_PER_TILE,
                                              ROWS_PER_TILE)])
    plsc.subcore_barrier()

    himask = jnp.full((16,), -65536, jnp.int32)

    def unpack(c):
        svs, dvs = [], []
        for g in range(NG):
            pk = pkx[c, pl.ds(g * 16, 16)]
            svs.append(lax.shift_right_logical(pk, 14))
            dvs.append(lax.bitwise_and(pk, PK - 1))
        return svs, dvs

    def write_src(c, su):
        svs, _ = unpack(c)
        for g in range(NG):
            su[pl.ds(g * 16, 16)] = svs[g]

    def process(c, rows, sbuf, su, du, gsem, ssem, wait_scatter,
                prefetch_guarded):
        pltpu.make_async_copy(hp_hbm.at[su], rows, gsem).wait()
        svs, dvs = unpack(c)
        ws = []
        for g in range(NG):
            pa_s = plsc.load_gather(apair_v, [svs[g]])
            pa_d = plsc.load_gather(apair_v, [dvs[g]])
            a_s = plsc.bitcast(lax.bitwise_and(pa_s, himask), jnp.float32)
            a_d = plsc.bitcast(lax.shift_left(pa_d, 16), jnp.float32)
            e = a_s + a_d
            e = jnp.where(e >= 0.0, e, 0.2 * e)
            ws.append(jnp.exp(e))
        if wait_scatter:
            pltpu.make_async_copy(sbuf, accum.at[du], ssem).wait()
        for g in range(NG):
            du[pl.ds(g * 16, 16)] = dvs[g]
        for g in range(NG):
            for l in range(16):
                wl = ws[g][l]
                i = g * 16 + l
                if bf16_rows:
                    for j in range(DB // 32):
                        v = plsc.bitcast(rows[i, pl.ds(j * 32, 32)],
                                         jnp.int32)
                        lo = plsc.bitcast(lax.shift_left(v, 16), jnp.float32)
                        sbuf[i, pl.ds(j * 32, 16)] = lo * wl
                        if j * 32 + 16 < D:
                            hi = plsc.bitcast(lax.bitwise_and(v, himask),
                                              jnp.float32)
                            sbuf[i, pl.ds(j * 32 + 16, 16)] = hi * wl
                else:
                    for j in range(D // 16):
                        sbuf[i, pl.ds(j * 16, 16)] = (
                            rows[i, pl.ds(j * 16, 16)] * wl)
        def prefetch():
            write_src(c + 2, su)
            pltpu.async_copy(hp_hbm.at[su], rows, gsem)
        if prefetch_guarded:
            pl.when(c + 2 < NCH)(prefetch)
        else:
            prefetch()
        pltpu.async_copy(sbuf, accum.at[du], ssem, add=True)

    write_src(0, su0)
    write_src(1, su1)
    pltpu.async_copy(hp_hbm.at[su0], rows0, g0)
    pltpu.async_copy(hp_hbm.at[su1], rows1, g1)
    process(0, rows0, sbuf0, su0, du0, g0, s0, False, False)
    process(1, rows1, sbuf1, su1, du1, g1, s1, False, False)

    def pair(k, carry):
        process(2 * k, rows0, sbuf0, su0, du0, g0, s0, True, True)
        process(2 * k + 1, rows1, sbuf1, su1, du1, g1, s1, True, True)
        return carry

    lax.fori_loop(1, NCH // 2, pair, 0)
    pltpu.make_async_copy(sbuf0, accum.at[du0], s0).wait()
    pltpu.make_async_copy(sbuf1, accum.at[du1], s1).wait()
    plsc.subcore_barrier()
    pltpu.sync_copy(
        accum.at[pl.ds(sid * ROWS_PER_TILE, ROWS_PER_TILE)],
        out_hbm.at[cid].at[pl.ds(sid * ROWS_PER_TILE, ROWS_PER_TILE)])


def _sc_edge(D, bf16_rows, hp, pk, apair):
    zeros = jnp.zeros((ROWS_PER_TILE, D), jnp.float32)
    rows_w = DB if bf16_rows else D
    rows_t = jnp.bfloat16 if bf16_rows else jnp.float32
    mesh = plsc.VectorSubcoreMesh(core_axis_name="c", subcore_axis_name="s")
    return pl.kernel(
        functools.partial(_sc_edge_body, D, bf16_rows),
        out_type=jax.ShapeDtypeStruct((2, N_PAD, D), jnp.float32),
        mesh=mesh,
        scratch_types=[
            pltpu.VMEM_SHARED((N_PAD, D), jnp.float32),
            pltpu.VMEM((CH, rows_w), rows_t),
            pltpu.VMEM((CH, rows_w), rows_t),
            pltpu.VMEM((CH, D), jnp.float32),
            pltpu.VMEM((CH, D), jnp.float32),
            pltpu.VMEM((NCH, CH), jnp.int32),
            pltpu.VMEM((CH,), jnp.int32),
            pltpu.VMEM((CH,), jnp.int32),
            pltpu.VMEM((CH,), jnp.int32),
            pltpu.VMEM((CH,), jnp.int32),
            pltpu.VMEM((N_PAD,), jnp.int32),
            pltpu.SemaphoreType.DMA,
            pltpu.SemaphoreType.DMA,
            pltpu.SemaphoreType.DMA,
            pltpu.SemaphoreType.DMA,
        ],
        compiler_params=pltpu.CompilerParams(needs_layout_passes=False,
                                             use_tc_tiling_on_sc=False),
    )(hp, pk, apair, zeros)


def kernel(x, edge_index, W1, att_src1, att_dst1, b1, W2, att_src2, att_dst2,
           b2):
    pad = jnp.full((E_PAD - E,), N * PK + N, jnp.int32)
    pk = jnp.concatenate(
        [(edge_index[0] * PK + edge_index[1]).astype(jnp.int32), pad]
    ).reshape(NW, NCH, CH)

    W1p = jnp.zeros((IN_DIM, D1), jnp.float32).at[:, :100].set(W1)
    att1T = jnp.zeros((D1, 2), jnp.float32)
    att1T = att1T.at[:100, 0].set(att_src1).at[:100, 1].set(att_dst1)
    b1p = jnp.zeros((1, D1), jnp.float32).at[0, :100].set(b1)
    W2p = jnp.zeros((D1, D2), jnp.float32).at[:100, :4].set(W2)
    att2T = jnp.zeros((D2, 2), jnp.float32)
    att2T = att2T.at[:4, 0].set(att_src2).at[:4, 1].set(att_dst2)
    b2p = jnp.zeros((1, D2), jnp.float32).at[0, :4].set(b2)

    perm = np.zeros((D1, DB), np.float32)
    for k in range(D1):
        j, t = k // 32, k % 32
        m = 32 * j + 2 * t if t < 16 else 32 * j + 2 * (t - 16) + 1
        perm[k, m] = 1.0
    Pmat = jnp.asarray(perm)

    def apack(a_s, a_d):
        bs = lax.bitcast_convert_type(a_s.astype(jnp.bfloat16), jnp.uint16)
        bd = lax.bitcast_convert_type(a_d.astype(jnp.bfloat16), jnp.uint16)
        return (bs.astype(jnp.int32) << 16) | bd.astype(jnp.int32)

    xp = jnp.zeros((N_PAD, IN_DIM), jnp.float32).at[:N].set(x)
    hp1, hpw1, av1 = _tc1(xp, W1p, att1T, Pmat)
    part1 = _sc_edge(D1, True, hp1, pk, apack(av1[:, 0], av1[:, 1]))
    hp2, hp2w, av2 = _tc2(part1[0], part1[1], hpw1, b1p, W2p, att2T)
    part2 = _sc_edge(D2, False, hp2, pk, apack(av2[:, 0], av2[:, 1]))
    (out,) = _tc3(part2[0], part2[1], hp2w, b2p)
    return out[:N]


# speedup vs baseline: 60.9544x; 1.0797x over previous
import functools

import jax
import jax.numpy as jnp
import numpy as np
from jax import lax
from jax.experimental import pallas as pl
from jax.experimental.pallas import tpu as pltpu
from jax.experimental.pallas import tpu_sc as plsc

N = 10000
N_PAD = 10112
E = 320000
IN_DIM = 128
D1 = 112
D2 = 16
ONES1 = 100
ONES2 = 4

NW = 32
CH = 96
NG = CH // 16
NCH = 106
E_PAD = NW * NCH * CH
DB = 128
CH2 = 128
NCH2 = 80
E_PAD2 = NW * NCH2 * CH2
PK = 16384
ROWS_PER_TILE = N_PAD // 16

BLK = 1264
GRID = N_PAD // BLK


def _leaky(x):
    return jnp.where(x >= 0.0, x, 0.2 * x)


def _tc1_body(x_ref, w_ref, attT_ref, pmat_ref, hp_ref, hpw_ref, av_ref):
    h = jnp.dot(x_ref[...], w_ref[...], preferred_element_type=jnp.float32)
    a2 = jnp.dot(h, attT_ref[...], preferred_element_type=jnp.float32)
    col = lax.broadcasted_iota(jnp.int32, (BLK, D1), 1)
    hp = jnp.where(col == ONES1, 1.0, h)
    wself = jnp.exp(_leaky(a2[:, 0:1] + a2[:, 1:2]))
    hps = jnp.dot(hp, pmat_ref[...], preferred_element_type=jnp.float32)
    hp_ref[...] = hps.astype(jnp.bfloat16)
    hpw_ref[...] = wself * hp
    av_ref[...] = a2


def _tc1(x, W1p, att1T, Pmat):
    return pl.pallas_call(
        _tc1_body,
        grid=(GRID,),
        in_specs=[
            pl.BlockSpec((BLK, IN_DIM), lambda i: (i, 0)),
            pl.BlockSpec((IN_DIM, D1), lambda i: (0, 0)),
            pl.BlockSpec((D1, 2), lambda i: (0, 0)),
            pl.BlockSpec((D1, DB), lambda i: (0, 0)),
        ],
        out_specs=[
            pl.BlockSpec((BLK, DB), lambda i: (i, 0)),
            pl.BlockSpec((BLK, D1), lambda i: (i, 0)),
            pl.BlockSpec((BLK, 2), lambda i: (i, 0)),
        ],
        out_shape=[
            jax.ShapeDtypeStruct((N_PAD, DB), jnp.bfloat16),
            jax.ShapeDtypeStruct((N_PAD, D1), jnp.float32),
            jax.ShapeDtypeStruct((N_PAD, 2), jnp.float32),
        ],
    )(x, W1p, att1T, Pmat)


def _tc2_body(p0_ref, p1_ref, hpw_ref, b1_ref, w2_ref, att2T_ref,
              hp2_ref, hp2w_ref, av2_ref):
    o = p0_ref[...] + p1_ref[...] + hpw_ref[...]
    denom = o[:, ONES1:ONES1 + 1] + 1e-16
    h1 = jnp.maximum(o / denom + b1_ref[...], 0.0)
    h2 = jnp.dot(h1, w2_ref[...], preferred_element_type=jnp.float32)
    a2 = jnp.dot(h2, att2T_ref[...], preferred_element_type=jnp.float32)
    col = lax.broadcasted_iota(jnp.int32, (BLK, D2), 1)
    hp2 = jnp.where(col == ONES2, 1.0, h2)
    wself = jnp.exp(_leaky(a2[:, 0:1] + a2[:, 1:2]))
    hp2_ref[...] = hp2
    hp2w_ref[...] = wself * hp2
    av2_ref[...] = a2


def _tc2(p0, p1, hpw1, b1p, W2p, att2T):
    return pl.pallas_call(
        _tc2_body,
        grid=(GRID,),
        in_specs=[
            pl.BlockSpec((BLK, D1), lambda i: (i, 0)),
            pl.BlockSpec((BLK, D1), lambda i: (i, 0)),
            pl.BlockSpec((BLK, D1), lambda i: (i, 0)),
            pl.BlockSpec((1, D1), lambda i: (0, 0)),
            pl.BlockSpec((D1, D2), lambda i: (0, 0)),
            pl.BlockSpec((D2, 2), lambda i: (0, 0)),
        ],
        out_specs=[
            pl.BlockSpec((BLK, D2), lambda i: (i, 0)),
            pl.BlockSpec((BLK, D2), lambda i: (i, 0)),
            pl.BlockSpec((BLK, 2), lambda i: (i, 0)),
        ],
        out_shape=[
            jax.ShapeDtypeStruct((N_PAD, D2), jnp.float32),
            jax.ShapeDtypeStruct((N_PAD, D2), jnp.float32),
            jax.ShapeDtypeStruct((N_PAD, 2), jnp.float32),
        ],
    )(p0, p1, hpw1, b1p, W2p, att2T)


def _tc3_body(q0_ref, q1_ref, hp2w_ref, b2_ref, out_ref):
    o = q0_ref[...] + q1_ref[...] + hp2w_ref[...]
    denom = o[:, ONES2:ONES2 + 1] + 1e-16
    logits = o / denom + b2_ref[...]
    col = lax.broadcasted_iota(jnp.int32, (BLK, D2), 1)
    valid = col < ONES2
    ml = jnp.where(valid, logits, -1e30)
    m = jnp.max(ml, axis=1, keepdims=True)
    s = jnp.sum(jnp.where(valid, jnp.exp(ml - m), 0.0), axis=1, keepdims=True)
    out_ref[...] = (logits - m - jnp.log(s))[:, 0:ONES2]


def _tc3(q0, q1, hp2w, b2p):
    return pl.pallas_call(
        _tc3_body,
        grid=(GRID,),
        in_specs=[
            pl.BlockSpec((BLK, D2), lambda i: (i, 0)),
            pl.BlockSpec((BLK, D2), lambda i: (i, 0)),
            pl.BlockSpec((BLK, D2), lambda i: (i, 0)),
            pl.BlockSpec((1, D2), lambda i: (0, 0)),
        ],
        out_specs=[pl.BlockSpec((BLK, ONES2), lambda i: (i, 0))],
        out_shape=[jax.ShapeDtypeStruct((N_PAD, ONES2), jnp.float32)],
    )(q0, q1, hp2w, b2p)


def _sc_edge_body(D, bf16_rows, hp_hbm, pk_hbm, apair_hbm, zeros_hbm,
                  out_hbm, accum, rows0, rows1, sbuf0, sbuf1, pkx,
                  su0, su1, du0, du1, apair_v, g0, g1, s0, s1):
    cid = lax.axis_index("c")
    sid = lax.axis_index("s")
    wid = sid * 2 + cid

    pltpu.sync_copy(pk_hbm.at[wid], pkx)
    pltpu.sync_copy(apair_hbm, apair_v)
    pltpu.sync_copy(zeros_hbm, accum.at[pl.ds(sid * ROWS_PER_TILE,
                                              ROWS_PER_TILE)])
    plsc.subcore_barrier()

    himask = jnp.full((16,), -65536, jnp.int32)

    def unpack(c):
        svs, dvs = [], []
        for g in range(NG):
            pk = pkx[c, pl.ds(g * 16, 16)]
            svs.append(lax.shift_right_logical(pk, 14))
            dvs.append(lax.bitwise_and(pk, PK - 1))
        return svs, dvs

    def write_src(c, su):
        svs, _ = unpack(c)
        for g in range(NG):
            su[pl.ds(g * 16, 16)] = svs[g]

    def process(c, rows, sbuf, su, du, gsem, ssem, wait_scatter,
                prefetch_guarded):
        pltpu.make_async_copy(hp_hbm.at[su], rows, gsem).wait()
        svs, dvs = unpack(c)
        ws = []
        for g in range(NG):
            pa_s = plsc.load_gather(apair_v, [svs[g]])
            pa_d = plsc.load_gather(apair_v, [dvs[g]])
            a_s = plsc.bitcast(lax.bitwise_and(pa_s, himask), jnp.float32)
            a_d = plsc.bitcast(lax.shift_left(pa_d, 16), jnp.float32)
            e = a_s + a_d
            e = jnp.where(e >= 0.0, e, 0.2 * e)
            ws.append(jnp.exp(e))
        if wait_scatter:
            pltpu.make_async_copy(sbuf, accum.at[du], ssem).wait()
        for g in range(NG):
            du[pl.ds(g * 16, 16)] = dvs[g]
        for g in range(NG):
            for l in range(16):
                wl = ws[g][l]
                i = g * 16 + l
                if bf16_rows:
                    for j in range(DB // 32):
                        v = plsc.bitcast(rows[i, pl.ds(j * 32, 32)],
                                         jnp.int32)
                        lo = plsc.bitcast(lax.shift_left(v, 16), jnp.float32)
                        sbuf[i, pl.ds(j * 32, 16)] = lo * wl
                        if j * 32 + 16 < D:
                            hi = plsc.bitcast(lax.bitwise_and(v, himask),
                                              jnp.float32)
                            sbuf[i, pl.ds(j * 32 + 16, 16)] = hi * wl
                else:
                    for j in range(D // 16):
                        sbuf[i, pl.ds(j * 16, 16)] = (
                            rows[i, pl.ds(j * 16, 16)] * wl)
        def prefetch():
            write_src(c + 2, su)
            pltpu.async_copy(hp_hbm.at[su], rows, gsem)
        if prefetch_guarded:
            pl.when(c + 2 < NCH)(prefetch)
        else:
            prefetch()
        pltpu.async_copy(sbuf, accum.at[du], ssem, add=True)

    write_src(0, su0)
    write_src(1, su1)
    pltpu.async_copy(hp_hbm.at[su0], rows0, g0)
    pltpu.async_copy(hp_hbm.at[su1], rows1, g1)
    process(0, rows0, sbuf0, su0, du0, g0, s0, False, False)
    process(1, rows1, sbuf1, su1, du1, g1, s1, False, False)

    def pair(k, carry):
        process(2 * k, rows0, sbuf0, su0, du0, g0, s0, True, True)
        process(2 * k + 1, rows1, sbuf1, su1, du1, g1, s1, True, True)
        return carry

    lax.fori_loop(1, NCH // 2, pair, 0)
    pltpu.make_async_copy(sbuf0, accum.at[du0], s0).wait()
    pltpu.make_async_copy(sbuf1, accum.at[du1], s1).wait()
    plsc.subcore_barrier()
    pltpu.sync_copy(
        accum.at[pl.ds(sid * ROWS_PER_TILE, ROWS_PER_TILE)],
        out_hbm.at[cid].at[pl.ds(sid * ROWS_PER_TILE, ROWS_PER_TILE)])


def _sc_edge(D, bf16_rows, hp, pk, apair):
    zeros = jnp.zeros((ROWS_PER_TILE, D), jnp.float32)
    rows_w = DB if bf16_rows else D
    rows_t = jnp.bfloat16 if bf16_rows else jnp.float32
    mesh = plsc.VectorSubcoreMesh(core_axis_name="c", subcore_axis_name="s")
    return pl.kernel(
        functools.partial(_sc_edge_body, D, bf16_rows),
        out_type=jax.ShapeDtypeStruct((2, N_PAD, D), jnp.float32),
        mesh=mesh,
        scratch_types=[
            pltpu.VMEM_SHARED((N_PAD, D), jnp.float32),
            pltpu.VMEM((CH, rows_w), rows_t),
            pltpu.VMEM((CH, rows_w), rows_t),
            pltpu.VMEM((CH, D), jnp.float32),
            pltpu.VMEM((CH, D), jnp.float32),
            pltpu.VMEM((NCH, CH), jnp.int32),
            pltpu.VMEM((CH,), jnp.int32),
            pltpu.VMEM((CH,), jnp.int32),
            pltpu.VMEM((CH,), jnp.int32),
            pltpu.VMEM((CH,), jnp.int32),
            pltpu.VMEM((N_PAD,), jnp.int32),
            pltpu.SemaphoreType.DMA,
            pltpu.SemaphoreType.DMA,
            pltpu.SemaphoreType.DMA,
            pltpu.SemaphoreType.DMA,
        ],
        compiler_params=pltpu.CompilerParams(needs_layout_passes=False,
                                             use_tc_tiling_on_sc=False),
    )(hp, pk, apair, zeros)


def _sc_edge2_body(hpT_hbm, pk_hbm, apair_hbm, zeros_hbm, out_hbm,
                   accum, sbuf0, sbuf1, pkx, du0, du1, cols_v, apair_v,
                   s0, s1):
    cid = lax.axis_index("c")
    sid = lax.axis_index("s")
    wid = sid * 2 + cid

    pltpu.sync_copy(pk_hbm.at[wid], pkx)
    pltpu.sync_copy(apair_hbm, apair_v)
    for j in range(ONES2):
        pltpu.sync_copy(hpT_hbm.at[j], cols_v.at[j])
    pltpu.sync_copy(zeros_hbm, accum.at[pl.ds(sid * ROWS_PER_TILE,
                                              ROWS_PER_TILE)])
    plsc.subcore_barrier()

    himask = jnp.full((16,), -65536, jnp.int32)
    NG2 = CH2 // 16

    def process(c, sbuf, du, ssem, wait_scatter, last):
        svs, dvs, ws = [], [], []
        for g in range(NG2):
            pk = pkx[c, pl.ds(g * 16, 16)]
            sv = lax.shift_right_logical(pk, 14)
            dv = lax.bitwise_and(pk, PK - 1)
            pa_s = plsc.load_gather(apair_v, [sv])
            pa_d = plsc.load_gather(apair_v, [dv])
            a_s = plsc.bitcast(lax.bitwise_and(pa_s, himask), jnp.float32)
            a_d = plsc.bitcast(lax.shift_left(pa_d, 16), jnp.float32)
            e = a_s + a_d
            e = jnp.where(e >= 0.0, e, 0.2 * e)
            svs.append(sv)
            dvs.append(dv)
            ws.append(jnp.exp(e))
        if wait_scatter:
            pltpu.make_async_copy(sbuf, accum.at[du], ssem).wait()
        rid = lax.iota(jnp.int32, 16)
        for g in range(NG2):
            du[pl.ds(g * 16, 16)] = dvs[g]
            rows16 = jnp.full((16,), g * 16, jnp.int32) + rid
            for j in range(ONES2):
                hv = plsc.load_gather(cols_v.at[j], [svs[g]])
                plsc.store_scatter(sbuf, [rows16,
                                          jnp.full((16,), j, jnp.int32)],
                                   hv * ws[g])
            plsc.store_scatter(sbuf, [rows16,
                                      jnp.full((16,), ONES2, jnp.int32)],
                               ws[g])
        pltpu.async_copy(sbuf, accum.at[du], ssem, add=True)

    process(0, sbuf0, du0, s0, False, False)
    process(1, sbuf1, du1, s1, False, False)

    def pair(k, carry):
        process(2 * k, sbuf0, du0, s0, True, False)
        process(2 * k + 1, sbuf1, du1, s1, True, False)
        return carry

    lax.fori_loop(1, NCH2 // 2, pair, 0)
    pltpu.make_async_copy(sbuf0, accum.at[du0], s0).wait()
    pltpu.make_async_copy(sbuf1, accum.at[du1], s1).wait()
    plsc.subcore_barrier()
    pltpu.sync_copy(
        accum.at[pl.ds(sid * ROWS_PER_TILE, ROWS_PER_TILE)],
        out_hbm.at[cid].at[pl.ds(sid * ROWS_PER_TILE, ROWS_PER_TILE)])


def _sc_edge2(hpT, pk2, apair):
    zeros = jnp.zeros((ROWS_PER_TILE, D2), jnp.float32)
    mesh = plsc.VectorSubcoreMesh(core_axis_name="c", subcore_axis_name="s")
    return pl.kernel(
        _sc_edge2_body,
        out_type=jax.ShapeDtypeStruct((2, N_PAD, D2), jnp.float32),
        mesh=mesh,
        scratch_types=[
            pltpu.VMEM_SHARED((N_PAD, D2), jnp.float32),
            pltpu.VMEM((CH2, D2), jnp.float32),
            pltpu.VMEM((CH2, D2), jnp.float32),
            pltpu.VMEM((NCH2, CH2), jnp.int32),
            pltpu.VMEM((CH2,), jnp.int32),
            pltpu.VMEM((CH2,), jnp.int32),
            pltpu.VMEM((ONES2, N_PAD), jnp.float32),
            pltpu.VMEM((N_PAD,), jnp.int32),
            pltpu.SemaphoreType.DMA,
            pltpu.SemaphoreType.DMA,
        ],
        compiler_params=pltpu.CompilerParams(needs_layout_passes=False,
                                             use_tc_tiling_on_sc=False),
    )(hpT, pk2, apair, zeros)


def kernel(x, edge_index, W1, att_src1, att_dst1, b1, W2, att_src2, att_dst2,
           b2):
    pke = (edge_index[0] * PK + edge_index[1]).astype(jnp.int32)
    pad = jnp.full((E_PAD - E,), N * PK + N, jnp.int32)
    pk = jnp.concatenate([pke, pad]).reshape(NW, NCH, CH)
    pad2 = jnp.full((E_PAD2 - E,), N * PK + N, jnp.int32)
    pk2 = jnp.concatenate([pke, pad2]).reshape(NW, NCH2, CH2)

    W1p = jnp.zeros((IN_DIM, D1), jnp.float32).at[:, :100].set(W1)
    att1T = jnp.zeros((D1, 2), jnp.float32)
    att1T = att1T.at[:100, 0].set(att_src1).at[:100, 1].set(att_dst1)
    b1p = jnp.zeros((1, D1), jnp.float32).at[0, :100].set(b1)
    W2p = jnp.zeros((D1, D2), jnp.float32).at[:100, :4].set(W2)
    att2T = jnp.zeros((D2, 2), jnp.float32)
    att2T = att2T.at[:4, 0].set(att_src2).at[:4, 1].set(att_dst2)
    b2p = jnp.zeros((1, D2), jnp.float32).at[0, :4].set(b2)

    perm = np.zeros((D1, DB), np.float32)
    for k in range(D1):
        j, t = k // 32, k % 32
        m = 32 * j + 2 * t if t < 16 else 32 * j + 2 * (t - 16) + 1
        perm[k, m] = 1.0
    Pmat = jnp.asarray(perm)

    def apack(a_s, a_d):
        bs = lax.bitcast_convert_type(a_s.astype(jnp.bfloat16), jnp.uint16)
        bd = lax.bitcast_convert_type(a_d.astype(jnp.bfloat16), jnp.uint16)
        return (bs.astype(jnp.int32) << 16) | bd.astype(jnp.int32)

    xp = jnp.zeros((N_PAD, IN_DIM), jnp.float32).at[:N].set(x)
    hp1, hpw1, av1 = _tc1(xp, W1p, att1T, Pmat)
    part1 = _sc_edge(D1, True, hp1, pk, apack(av1[:, 0], av1[:, 1]))
    hp2, hp2w, av2 = _tc2(part1[0], part1[1], hpw1, b1p, W2p, att2T)
    hpT = jnp.transpose(hp2[:, :ONES2])
    part2 = _sc_edge2(hpT, pk2, apack(av2[:, 0], av2[:, 1]))
    (out,) = _tc3(part2[0], part2[1], hp2w, b2p)
    return out[:N]
